# baseline (speedup 1.0000x reference)
import functools
import jax
import jax.numpy as jnp
from jax import lax
from jax.experimental import pallas as pl
from jax.experimental.pallas import tpu as pltpu
from jax.experimental.pallas import tpu_sc as plsc

VOCAB = 1000000
D = 64
B, L = 16384, 20
N = B * L
NC, NS, LANES = 2, 16, 16
NW = NC * NS
PER_W = N // NW
ROWS_W = B // NW
IDX_W = 128
GATHERS_W = PER_W // IDX_W
BN = 65536


def _tc_scores(vrow_ref, c_ref, tabT_ref, out_ref):
  res = jax.lax.dot_general(
      vrow_ref[...], tabT_ref[...], (((1,), (0,)), ((), ())),
      precision=jax.lax.Precision.HIGHEST,
      preferred_element_type=jnp.float32)
  out_ref[...] = res.reshape(BN) + c_ref[0, 0]


@jax.jit
def _scores(lan_emb, W_w, b_w, W_out, b_out):
  tabT = lan_emb.T
  vrow = jnp.dot(W_out, W_w, precision=jax.lax.Precision.HIGHEST)
  c = (jnp.dot(W_out, b_w.reshape(D, 1),
               precision=jax.lax.Precision.HIGHEST) + b_out).reshape(1, 1)
  grid = (VOCAB + BN - 1) // BN
  return pl.pallas_call(
      _tc_scores,
      grid=(grid,),
      in_specs=[
          pl.BlockSpec((1, D), lambda i: (0, 0)),
          pl.BlockSpec((1, 1), lambda i: (0, 0), memory_space=pltpu.SMEM),
          pl.BlockSpec((D, BN), lambda i: (0, i)),
      ],
      out_specs=pl.BlockSpec((BN,), lambda i: (i,)),
      out_shape=jax.ShapeDtypeStruct((VOCAB,), jnp.float32),
  )(vrow, c, tabT)


@functools.cache
def _build_sc_gather():
  mesh = plsc.VectorSubcoreMesh(
      core_axis_name="c", subcore_axis_name="s", num_cores=NC, num_subcores=NS)

  @functools.partial(
      pl.kernel,
      out_type=jax.ShapeDtypeStruct((L, B), jnp.float32),
      mesh=mesh,
      compiler_params=pltpu.CompilerParams(
          needs_layout_passes=False, use_tc_tiling_on_sc=False),
      scratch_types=[
          pltpu.VMEM((L, ROWS_W), jnp.int32),
          pltpu.VMEM((PER_W,), jnp.int32),
          pltpu.VMEM((PER_W,), jnp.float32),
          pltpu.VMEM((L, ROWS_W), jnp.float32),
          pltpu.SemaphoreType.DMA,
      ],
  )
  def _sc_gather(scores, featT, out, idx2_v, idx_v, tmp_v, out2_v, sem):
    cid = lax.axis_index("c")
    sid = lax.axis_index("s")
    wid = sid * NC + cid
    iota = lax.iota(jnp.int32, LANES)

    pltpu.sync_copy(featT.at[:, pl.ds(wid * ROWS_W, ROWS_W)], idx2_v)

    def flat_body(q, _):
      f = q * LANES + iota
      r = f // L
      c = f - r * L
      idx_v[pl.ds(q * LANES, LANES)] = plsc.load_gather(idx2_v, [c, r])
      return 0

    lax.fori_loop(0, PER_W // LANES, flat_body, 0)

    for k in range(GATHERS_W):
      pltpu.async_copy(
          scores.at[idx_v.at[pl.ds(k * IDX_W, IDX_W)]],
          tmp_v.at[pl.ds(k * IDX_W, IDX_W)],
          sem,
      )
    for k in range(GATHERS_W):
      pltpu.make_async_copy(
          scores.at[idx_v.at[pl.ds(k * IDX_W, IDX_W)]],
          tmp_v.at[pl.ds(k * IDX_W, IDX_W)],
          sem,
      ).wait()

    def out_body(q, _):
      f = q * LANES + iota
      r = f // L
      c = f - r * L
      plsc.store_scatter(out2_v, [c, r], tmp_v[pl.ds(q * LANES, LANES)])
      return 0

    lax.fori_loop(0, PER_W // LANES, out_body, 0)
    pltpu.sync_copy(out2_v, out.at[:, pl.ds(wid * ROWS_W, ROWS_W)])

  return _sc_gather


@jax.jit
def kernel(feature, lan_emb, W_w, b_w, W_out, b_out):
  s = _scores(lan_emb, W_w, b_w, W_out, b_out)
  return _build_sc_gather()(s, feature.astype(jnp.int32).T).T


# speedup vs baseline: 1.0054x; 1.0054x over previous
import functools
import jax
import jax.numpy as jnp
from jax import lax
from jax.experimental import pallas as pl
from jax.experimental.pallas import tpu as pltpu
from jax.experimental.pallas import tpu_sc as plsc

VOCAB = 1000000
D = 64
B, L = 16384, 20
N = B * L
NC, NS, LANES = 2, 16, 16
NW = NC * NS
PER_W = N // NW
ROWS_W = B // NW
IDX_W = 128
GATHERS_W = PER_W // IDX_W
BN = 32768


def _tc_scores(vrow_ref, c_ref, tabT_ref, out_ref):
  res = jax.lax.dot_general(
      vrow_ref[...], tabT_ref[...], (((1,), (0,)), ((), ())),
      precision=jax.lax.Precision.HIGHEST,
      preferred_element_type=jnp.float32)
  out_ref[...] = res.reshape(BN) + c_ref[0, 0]


@jax.jit
def _scores(lan_emb, W_w, b_w, W_out, b_out):
  tabT = lan_emb.T
  vrow = jnp.dot(W_out, W_w, precision=jax.lax.Precision.HIGHEST)
  c = (jnp.dot(W_out, b_w.reshape(D, 1),
               precision=jax.lax.Precision.HIGHEST) + b_out).reshape(1, 1)
  grid = (VOCAB + BN - 1) // BN
  return pl.pallas_call(
      _tc_scores,
      grid=(grid,),
      in_specs=[
          pl.BlockSpec((1, D), lambda i: (0, 0)),
          pl.BlockSpec((1, 1), lambda i: (0, 0), memory_space=pltpu.SMEM),
          pl.BlockSpec((D, BN), lambda i: (0, i)),
      ],
      out_specs=pl.BlockSpec((BN,), lambda i: (i,)),
      out_shape=jax.ShapeDtypeStruct((VOCAB,), jnp.float32),
  )(vrow, c, tabT)


@functools.cache
def _build_sc_gather():
  mesh = plsc.VectorSubcoreMesh(
      core_axis_name="c", subcore_axis_name="s", num_cores=NC, num_subcores=NS)

  @functools.partial(
      pl.kernel,
      out_type=jax.ShapeDtypeStruct((L, B), jnp.float32),
      mesh=mesh,
      compiler_params=pltpu.CompilerParams(
          needs_layout_passes=False, use_tc_tiling_on_sc=False),
      scratch_types=[
          pltpu.VMEM((L, ROWS_W), jnp.int32),
          pltpu.VMEM((PER_W,), jnp.int32),
          pltpu.VMEM((PER_W,), jnp.float32),
          pltpu.VMEM((L, ROWS_W), jnp.float32),
          pltpu.SemaphoreType.DMA,
      ],
  )
  def _sc_gather(scores, featT, out, idx2_v, idx_v, tmp_v, out2_v, sem):
    cid = lax.axis_index("c")
    sid = lax.axis_index("s")
    wid = sid * NC + cid
    iota = lax.iota(jnp.int32, LANES)

    pltpu.sync_copy(featT.at[:, pl.ds(wid * ROWS_W, ROWS_W)], idx2_v)

    def flat_body(q, _):
      f = q * LANES + iota
      r = f // L
      c = f - r * L
      idx_v[pl.ds(q * LANES, LANES)] = plsc.load_gather(idx2_v, [c, r])
      return 0

    lax.fori_loop(0, PER_W // LANES, flat_body, 0)

    for k in range(GATHERS_W):
      pltpu.async_copy(
          scores.at[idx_v.at[pl.ds(k * IDX_W, IDX_W)]],
          tmp_v.at[pl.ds(k * IDX_W, IDX_W)],
          sem,
      )
    for k in range(GATHERS_W):
      pltpu.make_async_copy(
          scores.at[idx_v.at[pl.ds(k * IDX_W, IDX_W)]],
          tmp_v.at[pl.ds(k * IDX_W, IDX_W)],
          sem,
      ).wait()

    def out_body(q, _):
      f = q * LANES + iota
      r = f // L
      c = f - r * L
      plsc.store_scatter(out2_v, [c, r], tmp_v[pl.ds(q * LANES, LANES)])
      return 0

    lax.fori_loop(0, PER_W // LANES, out_body, 0)
    pltpu.sync_copy(out2_v, out.at[:, pl.ds(wid * ROWS_W, ROWS_W)])

  return _sc_gather


@jax.jit
def kernel(feature, lan_emb, W_w, b_w, W_out, b_out):
  s = _scores(lan_emb, W_w, b_w, W_out, b_out)
  return _build_sc_gather()(s, feature.astype(jnp.int32).T).T
